# baseline (speedup 1.0000x reference)
import jax
import jax.numpy as jnp
from jax.experimental import pallas as pl
from jax.experimental.pallas import tpu as pltpu

_MIB = 1024 * 1024


def _make_body(B, s):
    def _body(x_ref, w_ref, b_ref, o_ref):
        _, Hb, W, C = x_ref.shape
        Ho, Wo = Hb // s, W // s
        w = w_ref[...]
        b = b_ref[...]
        for i in range(B):
            acc = x_ref[i, pl.ds(0, Ho, stride=s), pl.ds(0, Wo, stride=s), :]
            for dy in range(s):
                for dx in range(s):
                    if dy == 0 and dx == 0:
                        continue
                    acc = acc + x_ref[i, pl.ds(dy, Ho, stride=s),
                                      pl.ds(dx, Wo, stride=s), :]
            y = jnp.dot(acc.reshape(Ho * Wo, C), w,
                        preferred_element_type=jnp.float32) + b
            o_ref[i] = y.reshape(Ho, Wo, w.shape[1])
    return _body


def kernel(x_nchw, conv_weight, conv_bias):
    N, C_in, H, W = x_nchw.shape
    s = 2
    H_out, W_out = H // s, W // s
    w = jnp.asarray(conv_weight)
    if w.ndim == 4:
        w = w[:, :, 0, 0]
    C_out = w.shape[0]

    B = 1
    x_nhwc = jnp.transpose(x_nchw, (0, 2, 3, 1))
    w_t = (jnp.transpose(w) / float(s * s)).astype(jnp.float32)
    b_row = jnp.asarray(conv_bias).astype(jnp.float32).reshape(1, C_out)

    out_nhwc = pl.pallas_call(
        _make_body(B, s),
        out_shape=jax.ShapeDtypeStruct((N, H_out, W_out, C_out), x_nchw.dtype),
        grid=(N // B,),
        in_specs=[
            pl.BlockSpec((B, H, W, C_in), lambda nb: (nb, 0, 0, 0)),
            pl.BlockSpec((C_in, C_out), lambda nb: (0, 0)),
            pl.BlockSpec((1, C_out), lambda nb: (0, 0)),
        ],
        out_specs=pl.BlockSpec((B, H_out, W_out, C_out), lambda nb: (nb, 0, 0, 0)),
        compiler_params=pltpu.CompilerParams(
            dimension_semantics=("parallel",),
            vmem_limit_bytes=64 * _MIB,
        ),
    )(x_nhwc, w_t, b_row)
    return jnp.transpose(out_nhwc, (0, 3, 1, 2))


# speedup vs baseline: 1.2811x; 1.2811x over previous
import jax
import jax.numpy as jnp
from jax.experimental import pallas as pl
from jax.experimental.pallas import tpu as pltpu

_MIB = 1024 * 1024


def _make_body(B, s):
    def _body(x_ref, w_ref, b_ref, o_ref):
        _, Hb, W, C = x_ref.shape
        Ho, Wo = Hb // s, W // s
        w = w_ref[...]
        b = b_ref[...]
        for i in range(B):
            acc = x_ref[i, pl.ds(0, Ho, stride=s), pl.ds(0, Wo, stride=s), :]
            for dy in range(s):
                for dx in range(s):
                    if dy == 0 and dx == 0:
                        continue
                    acc = acc + x_ref[i, pl.ds(dy, Ho, stride=s),
                                      pl.ds(dx, Wo, stride=s), :]
            y = jnp.dot(acc.reshape(Ho * Wo, C), w,
                        preferred_element_type=jnp.float32) + b
            o_ref[i] = y.reshape(Ho, Wo, w.shape[1])
    return _body


def kernel(x_nchw, conv_weight, conv_bias):
    N, C_in, H, W = x_nchw.shape
    s = 2
    H_out, W_out = H // s, W // s
    w = jnp.asarray(conv_weight)
    if w.ndim == 4:
        w = w[:, :, 0, 0]
    C_out = w.shape[0]

    B = 4
    x_nhwc = jnp.transpose(x_nchw, (0, 2, 3, 1))
    w_t = (jnp.transpose(w) / float(s * s)).astype(jnp.float32)
    b_row = jnp.asarray(conv_bias).astype(jnp.float32).reshape(1, C_out)

    out_nhwc = pl.pallas_call(
        _make_body(B, s),
        out_shape=jax.ShapeDtypeStruct((N, H_out, W_out, C_out), x_nchw.dtype),
        grid=(N // B,),
        in_specs=[
            pl.BlockSpec((B, H, W, C_in), lambda nb: (nb, 0, 0, 0)),
            pl.BlockSpec((C_in, C_out), lambda nb: (0, 0)),
            pl.BlockSpec((1, C_out), lambda nb: (0, 0)),
        ],
        out_specs=pl.BlockSpec((B, H_out, W_out, C_out), lambda nb: (nb, 0, 0, 0)),
        compiler_params=pltpu.CompilerParams(
            dimension_semantics=("parallel",),
            vmem_limit_bytes=64 * _MIB,
        ),
    )(x_nhwc, w_t, b_row)
    return jnp.transpose(out_nhwc, (0, 3, 1, 2))
